# baseline (device time: 14993 ns/iter reference)
import jax
import jax.numpy as jnp
from jax import lax
from jax.experimental import pallas as pl
from jax.experimental.pallas import tpu as pltpu

N_DEV = 8
B, SQ, D = 2, 128, 512
HQ_PER = 8
DH = 64
BSQ = B * SQ
MASKS = (1, 3, 4)
NC = 16
CPB = NC // B
CROWS = BSQ // NC


def kernel(x, Wq, Wo, Wk, Wv):
    def body(x_hbm, wq_hbm, wo_hbm, wk_hbm, wv_hbm, out_ref,
             x_v, wq_v, wo_v, wk_v, wv_v,
             load_sems, send_ref, recv_ref, send_sems, recv_sems):
        my = lax.axis_index("i")

        kv_start = my * (2 * DH)
        half = (HQ_PER * DH) // 2
        loads = [
            pltpu.make_async_copy(x_hbm, x_v, load_sems.at[0]),
            pltpu.make_async_copy(wq_hbm.at[:, pl.ds(0, half)],
                                  wq_v.at[:, pl.ds(0, half)], load_sems.at[1]),
            pltpu.make_async_copy(wk_hbm.at[:, pl.ds(kv_start, 2 * DH)],
                                  wk_v, load_sems.at[2]),
            pltpu.make_async_copy(wv_hbm.at[:, pl.ds(kv_start, 2 * DH)],
                                  wv_v, load_sems.at[3]),
            pltpu.make_async_copy(wo_hbm, wo_v, load_sems.at[4]),
            pltpu.make_async_copy(wq_hbm.at[:, pl.ds(half, half)],
                                  wq_v.at[:, pl.ds(half, half)],
                                  load_sems.at[5]),
        ]
        for ld in loads:
            ld.start()

        partners = [jnp.bitwise_xor(my, mk) for mk in MASKS]
        barrier_sem = pltpu.get_barrier_semaphore()
        for p in partners:
            pl.semaphore_signal(barrier_sem, inc=1, device_id=(p,),
                                device_id_type=pl.DeviceIdType.MESH)

        loads[0].wait()
        loads[1].wait()
        x2 = x_v[...].reshape(BSQ, D).astype(jnp.bfloat16)
        q0 = jnp.dot(x2, wq_v[:, :half].astype(jnp.bfloat16),
                     preferred_element_type=jnp.float32).astype(jnp.bfloat16)
        loads[5].wait()
        q1 = jnp.dot(x2, wq_v[:, half:].astype(jnp.bfloat16),
                     preferred_element_type=jnp.float32).astype(jnp.bfloat16)
        q = jnp.concatenate([q0, q1], axis=1)

        loads[2].wait()
        loads[3].wait()
        k = jnp.dot(x2, wk_v[...].astype(jnp.bfloat16),
                    preferred_element_type=jnp.float32).astype(jnp.bfloat16)
        v = jnp.dot(x2, wv_v[...].astype(jnp.bfloat16),
                    preferred_element_type=jnp.float32).astype(jnp.bfloat16)

        def start_rdma(r, c, data):
            send_ref[r, c] = data.astype(jnp.bfloat16)
            rdma = pltpu.make_async_remote_copy(
                src_ref=send_ref.at[r, c],
                dst_ref=recv_ref.at[r, c],
                send_sem=send_sems.at[r, c],
                recv_sem=recv_sems.at[r, c],
                device_id=(partners[(r + c) % 3],),
                device_id_type=pl.DeviceIdType.MESH,
            )
            rdma.start()
            return rdma

        accs = [None] * NC
        rdmas = [[None] * NC for _ in range(3)]
        for b in range(B):
            r0 = b * SQ
            head_ctx = [None] * HQ_PER
            for kvh in range(2):
                qs = jnp.concatenate(
                    [q[r0:r0 + SQ, (4 * kvh + j) * DH:(4 * kvh + j + 1) * DH]
                     for j in range(4)], axis=0)
                kh = k[r0:r0 + SQ, kvh * DH:(kvh + 1) * DH]
                vh = v[r0:r0 + SQ, kvh * DH:(kvh + 1) * DH]
                s = jnp.dot(qs, kh.T, preferred_element_type=jnp.float32) * 0.125
                m = jnp.max(s, axis=-1, keepdims=True)
                p = jnp.exp(s - m)
                l = jnp.sum(p, axis=-1, keepdims=True)
                o = jnp.dot(p.astype(jnp.bfloat16), vh,
                            preferred_element_type=jnp.float32) / l
                o = o.astype(jnp.bfloat16)
                for j in range(4):
                    head_ctx[4 * kvh + j] = o[j * SQ:(j + 1) * SQ, :]
            ctx_b = jnp.concatenate(head_ctx, axis=1)
            if b == 0:
                loads[4].wait()
            partial_b = jnp.dot(ctx_b, wo_v[...].astype(jnp.bfloat16),
                                preferred_element_type=jnp.float32)
            if b == 0:
                pl.semaphore_wait(barrier_sem, 3)
            for h in range(CPB):
                c = b * CPB + h
                accs[c] = partial_b[h * CROWS:(h + 1) * CROWS, :]
                rdmas[0][c] = start_rdma(0, c, accs[c])

        for r in range(1, 3):
            for c in range(NC):
                rdmas[r - 1][c].wait()
                accs[c] = accs[c] + recv_ref[r - 1, c].astype(jnp.float32)
                rdmas[r][c] = start_rdma(r, c, accs[c])
        for c in range(NC):
            rdmas[2][c].wait()
            accs[c] = accs[c] + recv_ref[2, c].astype(jnp.float32)
            b, h = c // CPB, c % CPB
            out_ref[b, pl.ds(h * CROWS, CROWS), :] = accs[c].astype(jnp.bfloat16)

    return pl.pallas_call(
        body,
        out_shape=jax.ShapeDtypeStruct((B, SQ, D), jnp.bfloat16),
        in_specs=[pl.BlockSpec(memory_space=pltpu.MemorySpace.HBM)] * 5,
        out_specs=pl.BlockSpec(memory_space=pltpu.VMEM),
        scratch_shapes=[
            pltpu.VMEM((B, SQ, D), jnp.float32),
            pltpu.VMEM((D, HQ_PER * DH), jnp.float32),
            pltpu.VMEM((HQ_PER * DH, D), jnp.float32),
            pltpu.VMEM((D, 2 * DH), jnp.float32),
            pltpu.VMEM((D, 2 * DH), jnp.float32),
            pltpu.SemaphoreType.DMA((6,)),
            pltpu.VMEM((3, NC, BSQ // NC, D), jnp.bfloat16),
            pltpu.VMEM((3, NC, BSQ // NC, D), jnp.bfloat16),
            pltpu.SemaphoreType.DMA((3, NC)),
            pltpu.SemaphoreType.DMA((3, NC)),
        ],
        compiler_params=pltpu.CompilerParams(collective_id=0),
    )(*(pltpu.with_memory_space_constraint(a, pltpu.MemorySpace.HBM)
        for a in (x, Wq, Wo, Wk, Wv)))


# device time: 14643 ns/iter; 1.0239x vs baseline; 1.0239x over previous
import jax
import jax.numpy as jnp
from jax import lax
from jax.experimental import pallas as pl
from jax.experimental.pallas import tpu as pltpu

N_DEV = 8
B, SQ, D = 2, 128, 512
HQ_PER = 8
DH = 64
BSQ = B * SQ
MASKS = (1, 3, 4)
NC = 16
CPB = NC // B
CROWS = BSQ // NC


def kernel(x, Wq, Wo, Wk, Wv):
    def body(x_hbm, wq_hbm, wo_hbm, wk_hbm, wv_hbm, out_ref,
             x_v, wq_v, wo_v, wk_v, wv_v,
             load_sems, send_ref, recv_ref, send_sems, recv_sems):
        my = lax.axis_index("i")

        kv_start = my * (2 * DH)
        loads = [
            pltpu.make_async_copy(x_hbm, x_v, load_sems.at[0]),
            pltpu.make_async_copy(wq_hbm, wq_v, load_sems.at[1]),
            pltpu.make_async_copy(wk_hbm.at[:, pl.ds(kv_start, 2 * DH)],
                                  wk_v, load_sems.at[2]),
            pltpu.make_async_copy(wv_hbm.at[:, pl.ds(kv_start, 2 * DH)],
                                  wv_v, load_sems.at[3]),
            pltpu.make_async_copy(wo_hbm, wo_v, load_sems.at[4]),
        ]
        for ld in loads:
            ld.start()

        partners = [jnp.bitwise_xor(my, mk) for mk in MASKS]
        barrier_sem = pltpu.get_barrier_semaphore()
        for p in partners:
            pl.semaphore_signal(barrier_sem, inc=1, device_id=(p,),
                                device_id_type=pl.DeviceIdType.MESH)

        loads[0].wait()
        loads[1].wait()
        x2 = x_v[...].reshape(BSQ, D).astype(jnp.bfloat16)
        q = jnp.dot(x2, wq_v[...].astype(jnp.bfloat16),
                    preferred_element_type=jnp.float32)
        q = q.astype(jnp.bfloat16)

        loads[2].wait()
        loads[3].wait()
        k = jnp.dot(x2, wk_v[...].astype(jnp.bfloat16),
                    preferred_element_type=jnp.float32).astype(jnp.bfloat16)
        v = jnp.dot(x2, wv_v[...].astype(jnp.bfloat16),
                    preferred_element_type=jnp.float32).astype(jnp.bfloat16)

        def start_rdma(r, c, data):
            send_ref[r, c] = data.astype(jnp.bfloat16)
            rdma = pltpu.make_async_remote_copy(
                src_ref=send_ref.at[r, c],
                dst_ref=recv_ref.at[r, c],
                send_sem=send_sems.at[r, c],
                recv_sem=recv_sems.at[r, c],
                device_id=(partners[(r + c) % 3],),
                device_id_type=pl.DeviceIdType.MESH,
            )
            rdma.start()
            return rdma

        accs = [None] * NC
        rdmas = [[None] * NC for _ in range(3)]
        for b in range(B):
            r0 = b * SQ
            head_ctx = [None] * HQ_PER
            for kvh in range(2):
                qs = jnp.concatenate(
                    [q[r0:r0 + SQ, (4 * kvh + j) * DH:(4 * kvh + j + 1) * DH]
                     for j in range(4)], axis=0)
                kh = k[r0:r0 + SQ, kvh * DH:(kvh + 1) * DH]
                vh = v[r0:r0 + SQ, kvh * DH:(kvh + 1) * DH]
                s = jnp.dot(qs, kh.T, preferred_element_type=jnp.float32) * 0.125
                m = jnp.max(s, axis=-1, keepdims=True)
                p = jnp.exp(s - m)
                l = jnp.sum(p, axis=-1, keepdims=True)
                o = jnp.dot(p.astype(jnp.bfloat16), vh,
                            preferred_element_type=jnp.float32) / l
                o = o.astype(jnp.bfloat16)
                for j in range(4):
                    head_ctx[4 * kvh + j] = o[j * SQ:(j + 1) * SQ, :]
            ctx_b = jnp.concatenate(head_ctx, axis=1)
            if b == 0:
                loads[4].wait()
            partial_b = jnp.dot(ctx_b, wo_v[...].astype(jnp.bfloat16),
                                preferred_element_type=jnp.float32)
            if b == 0:
                pl.semaphore_wait(barrier_sem, 3)
            for h in range(CPB):
                c = b * CPB + h
                accs[c] = partial_b[h * CROWS:(h + 1) * CROWS, :]
                rdmas[0][c] = start_rdma(0, c, accs[c])

        for r in range(1, 3):
            for c in range(NC):
                rdmas[r - 1][c].wait()
                accs[c] = accs[c] + recv_ref[r - 1, c].astype(jnp.float32)
                rdmas[r][c] = start_rdma(r, c, accs[c])
        for c in range(NC):
            rdmas[2][c].wait()
            accs[c] = accs[c] + recv_ref[2, c].astype(jnp.float32)
            b, h = c // CPB, c % CPB
            out_ref[b, pl.ds(h * CROWS, CROWS), :] = accs[c].astype(jnp.bfloat16)

    return pl.pallas_call(
        body,
        out_shape=jax.ShapeDtypeStruct((B, SQ, D), jnp.bfloat16),
        in_specs=[pl.BlockSpec(memory_space=pltpu.MemorySpace.HBM)] * 5,
        out_specs=pl.BlockSpec(memory_space=pltpu.VMEM),
        scratch_shapes=[
            pltpu.VMEM((B, SQ, D), jnp.float32),
            pltpu.VMEM((D, HQ_PER * DH), jnp.float32),
            pltpu.VMEM((HQ_PER * DH, D), jnp.float32),
            pltpu.VMEM((D, 2 * DH), jnp.float32),
            pltpu.VMEM((D, 2 * DH), jnp.float32),
            pltpu.SemaphoreType.DMA((5,)),
            pltpu.VMEM((3, NC, BSQ // NC, D), jnp.bfloat16),
            pltpu.VMEM((3, NC, BSQ // NC, D), jnp.bfloat16),
            pltpu.SemaphoreType.DMA((3, NC)),
            pltpu.SemaphoreType.DMA((3, NC)),
        ],
        compiler_params=pltpu.CompilerParams(collective_id=0),
    )(*(pltpu.with_memory_space_constraint(a, pltpu.MemorySpace.HBM)
        for a in (x, Wq, Wo, Wk, Wv)))
